# baseline (speedup 1.0000x reference)
import functools

import jax
import jax.numpy as jnp
import numpy as np
from jax import lax
from jax.experimental import pallas as pl
from jax.experimental.pallas import tpu as pltpu
from jax.experimental.pallas import tpu_sc as plsc

_B = 64
_P = 32768
_K = 8192
_L = 16
_NC = 2
_NS = 16
_NW = _NC * _NS
_RPW = _B // _NW
_NV0 = _P // _L
_PITCH = 257
_MIN32 = np.int32(-2147483648)


def _keys(x):
    i = lax.bitcast_convert_type(x, jnp.int32)
    return i ^ ((i >> 31) | _MIN32)


def _byte(u, shift):
    return lax.shift_right_logical(u, shift) & 255


def _clear_hist(hist):
    zeros = jnp.zeros((_L,), jnp.int32)

    def clr(j, carry):
        hist[pl.ds(j * _L, _L)] = zeros
        return carry

    lax.fori_loop(0, _PITCH, clr, 0, unroll=16)


def _select(hist, tot, need, iota):
    for c in range(16):
        tc = hist[pl.ds(c * _L, _L)]
        for lane in range(1, _L):
            tc = tc + hist[pl.ds(lane * _PITCH + c * _L, _L)]
        tot[pl.ds(c * _L, _L)] = tc

    def cscan(i, carry):
        s_run, c_sel, s_above = carry
        cc = 15 - i
        csum = jnp.sum(tot[pl.ds(cc * _L, _L)])
        s_new = s_run + csum
        hit = jnp.logical_and(c_sel < 0, s_new >= need)
        c_sel = jnp.where(hit, cc, c_sel)
        s_above = jnp.where(hit, s_run, s_above)
        return s_new, c_sel, s_above

    _, c_sel, s_above = lax.fori_loop(
        0, 16, cscan, (jnp.int32(0), jnp.int32(-1), jnp.int32(0)))

    tcv = tot[pl.ds(c_sel * _L, _L)]
    suf = lax.rev(plsc.cumsum(lax.rev(tcv, (0,))), (0,))
    mask = (s_above + suf) >= need
    j_sel = jnp.sum(mask.astype(jnp.int32)) - 1
    sel = iota == j_sel
    cnt_ge = s_above + jnp.sum(jnp.where(sel, suf, 0))
    n_in = jnp.sum(jnp.where(sel, tcv, 0))
    cnt_gt = cnt_ge - n_in
    return c_sel * _L + j_sel, need - cnt_gt, n_in


def _process_row(row, cand, hist, tot, acc, iota, lane_base):
    need = jnp.int32(_K)
    ones = jnp.ones((_L,), jnp.int32)

    with jax.named_scope("ph_clear0"):
        _clear_hist(hist)

    @plsc.parallel_loop(0, _NV0, unroll=8)
    def _hist0(i):
        x = row[pl.ds(i * _L, _L)]
        b = _byte(_keys(x), 24)
        plsc.addupdate_scatter(hist, [lane_base + b], ones)

    with jax.named_scope("ph_sel0"):
        b0, need, _ = _select(hist, tot, need, iota)

    def body0(i, carry):
        acc, cnt16 = carry
        xs = [row[pl.ds((i * 16 + k) * _L, _L)] for k in range(16)]
        bs_ = [_byte(_keys(x), 24) for x in xs]
        for x, b in zip(xs, bs_):
            acc = acc + jnp.where(b > b0, x, 0.0)
            match = b == b0
            plsc.store_scatter(cand, [cnt16 + iota], x, mask=match)
            cnt16 = cnt16 + jnp.where(match, 16, 0)
        return acc, cnt16

    with jax.named_scope("ph_compact0"):
        acc, cnt16 = lax.fori_loop(
            0, _NV0 // 16, body0, (acc, jnp.zeros((_L,), jnp.int32)))
    cnt = cnt16 >> 4


    src, dst = cand, row
    u_t = b0 << 24
    for lvl in (1, 2):
        shift = 24 - 8 * lvl
        with jax.named_scope(f"ph_l{lvl}max"):
            mv = jnp.max(cnt)
        with jax.named_scope(f"ph_l{lvl}clear"):
            _clear_hist(hist)

        def bodyh(j, carry, src=src, cnt=cnt, shift=shift):
            x = src[pl.ds(j * _L, _L)]
            b = _byte(_keys(x), shift)
            plsc.addupdate_scatter(hist, [lane_base + b], ones, mask=j < cnt)
            return carry

        with jax.named_scope(f"ph_l{lvl}hist"):
            lax.fori_loop(0, mv, bodyh, 0)
        with jax.named_scope(f"ph_l{lvl}sel"):
            bs, need, _ = _select(hist, tot, need, iota)

        def bodyc(j, carry, src=src, dst=dst, cnt=cnt, shift=shift, bs=bs):
            acc, cnt16 = carry
            x = src[pl.ds(j * _L, _L)]
            b = _byte(_keys(x), shift)
            valid = j < cnt
            acc = acc + jnp.where(jnp.logical_and(valid, b > bs), x, 0.0)
            match = jnp.logical_and(valid, b == bs)
            plsc.store_scatter(dst, [cnt16 + iota], x, mask=match)
            cnt16 = cnt16 + jnp.where(match, 16, 0)
            return acc, cnt16

        with jax.named_scope(f"ph_l{lvl}comp"):
            acc, cnt16 = lax.fori_loop(
                0, mv, bodyc, (acc, jnp.zeros((_L,), jnp.int32)))
        cnt = cnt16 >> 4
        u_t = u_t | (bs << shift)
        src, dst = dst, src

    scope_rest = jax.named_scope("ph_l3"); scope_rest.__enter__()
    mv = jnp.max(cnt)
    _clear_hist(hist)

    def bodyh3(j, carry, src=src, cnt=cnt):
        x = src[pl.ds(j * _L, _L)]
        b = _byte(_keys(x), 0)
        plsc.addupdate_scatter(hist, [lane_base + b], ones, mask=j < cnt)
        return carry

    lax.fori_loop(0, mv, bodyh3, 0)
    b3, need, _ = _select(hist, tot, need, iota)

    def body3(j, acc, src=src, cnt=cnt, b3=b3):
        x = src[pl.ds(j * _L, _L)]
        b = _byte(_keys(x), 0)
        above = jnp.logical_and(j < cnt, b > b3)
        return acc + jnp.where(above, x, 0.0)

    acc = lax.fori_loop(0, mv, body3, acc)

    scope_rest.__exit__(None, None, None)
    u_t = u_t | b3
    uv = jnp.zeros((_L,), jnp.int32) + u_t
    iv = jnp.where(uv < 0, uv ^ _MIN32, ~uv)
    tv = lax.bitcast_convert_type(iv, jnp.float32)
    needf = (jnp.zeros((_L,), jnp.int32) + need).astype(jnp.float32)
    return acc + jnp.where(iota == 0, needf * tv, 0.0)


def _sc_body(loss_hbm, out_hbm, row0, cand, hist, tot, accv, sem0):
    wid = lax.axis_index("s") * _NC + lax.axis_index("c")
    iota = lax.iota(jnp.int32, _L)
    lane_base = iota * _PITCH

    pltpu.make_async_copy(loss_hbm.at[wid], row0, sem0).start()

    acc = jnp.zeros((_L,), jnp.float32)
    pltpu.make_async_copy(loss_hbm.at[wid], row0, sem0).wait()
    acc = _process_row(row0, cand, hist, tot, acc, iota, lane_base)

    accv[...] = acc
    pltpu.sync_copy(accv, out_hbm.at[wid])


@functools.partial(
    pl.kernel,
    out_type=jax.ShapeDtypeStruct((_NW, _L), jnp.float32),
    mesh=plsc.VectorSubcoreMesh(core_axis_name="c", subcore_axis_name="s"),
    compiler_params=pltpu.CompilerParams(needs_layout_passes=False),
    scratch_types=[
        pltpu.VMEM((_P,), jnp.float32),
        pltpu.VMEM((_P,), jnp.float32),
        pltpu.VMEM((_L * _PITCH,), jnp.int32),
        pltpu.VMEM((256,), jnp.int32),
        pltpu.VMEM((_L,), jnp.float32),
        pltpu.SemaphoreType.DMA,
    ],
)
def _sc_kernel(loss_hbm, out_hbm, *scratch):
    _sc_body(loss_hbm, out_hbm, *scratch)


def _tc_body(x_ref, out_ref):
    x = x_ref[...]
    i = lax.bitcast_convert_type(x, jnp.int32)
    s = i ^ ((i >> 31) & jnp.int32(0x7FFFFFFF))

    def step(_, lohi):
        lo, hi = lohi
        mid = (lo >> 1) + (hi >> 1) + (lo & hi & 1) + ((lo ^ hi) & 1)
        cnt = jnp.sum((s >= mid).astype(jnp.int32), axis=1, keepdims=True)
        pred = cnt >= _K
        lo = jnp.where(pred, mid, lo)
        hi = jnp.where(pred, hi, mid - 1)
        return lo, hi

    nrows = x.shape[0]
    lo0 = jnp.full((nrows, 1), jnp.iinfo(jnp.int32).min, jnp.int32)
    hi0 = jnp.full((nrows, 1), jnp.iinfo(jnp.int32).max, jnp.int32)
    t, _ = jax.lax.fori_loop(0, 32, step, (lo0, hi0))

    gt = s > t
    cnt_gt = jnp.sum(gt.astype(jnp.float32), axis=1, keepdims=True)
    sum_gt = jnp.sum(jnp.where(gt, x, 0.0), axis=1, keepdims=True)
    tf = lax.bitcast_convert_type(t ^ ((t >> 31) & jnp.int32(0x7FFFFFFF)),
                                  jnp.float32)
    row = sum_gt + (jnp.float32(_K) - cnt_gt) * tf
    out_ref[0, 0] = jnp.sum(row)


def _tc_sum(x):
    return pl.pallas_call(
        _tc_body,
        grid=(1,),
        out_shape=jax.ShapeDtypeStruct((1, 1), jnp.float32),
        in_specs=[pl.BlockSpec((_B - _NW, _P), lambda i: (1, 0))],
        out_specs=pl.BlockSpec((1, 1), lambda i: (0, 0),
                               memory_space=pltpu.SMEM),
    )(x)


def kernel(loss, dummy):
    sc_part = _sc_kernel(loss)
    tc_part = _tc_sum(loss)
    return (jnp.sum(sc_part) + tc_part[0, 0]) / jnp.float32(_B * _K)


# speedup vs baseline: 1.2831x; 1.2831x over previous
import functools

import jax
import jax.numpy as jnp
import numpy as np
from jax import lax
from jax.experimental import pallas as pl
from jax.experimental.pallas import tpu as pltpu
from jax.experimental.pallas import tpu_sc as plsc

_B = 64
_P = 32768
_K = 8192
_L = 16
_NC = 2
_NS = 16
_NW = _NC * _NS
_RPW = _B // _NW
_NV0 = _P // _L
_PITCH = 257
_MIN32 = np.int32(-2147483648)


def _keys(x):
    i = lax.bitcast_convert_type(x, jnp.int32)
    return i ^ ((i >> 31) | _MIN32)


def _byte(u, shift):
    return lax.shift_right_logical(u, shift) & 255


def _clear_hist(hist):
    zeros = jnp.zeros((_L,), jnp.int32)

    def clr(j, carry):
        hist[pl.ds(j * _L, _L)] = zeros
        return carry

    lax.fori_loop(0, _PITCH, clr, 0, unroll=16)


def _select(hist, tot, need, iota):
    for c in range(16):
        tc = hist[pl.ds(c * _L, _L)]
        for lane in range(1, _L):
            tc = tc + hist[pl.ds(lane * _PITCH + c * _L, _L)]
        tot[pl.ds(c * _L, _L)] = tc

    def cscan(i, carry):
        s_run, c_sel, s_above = carry
        cc = 15 - i
        csum = jnp.sum(tot[pl.ds(cc * _L, _L)])
        s_new = s_run + csum
        hit = jnp.logical_and(c_sel < 0, s_new >= need)
        c_sel = jnp.where(hit, cc, c_sel)
        s_above = jnp.where(hit, s_run, s_above)
        return s_new, c_sel, s_above

    _, c_sel, s_above = lax.fori_loop(
        0, 16, cscan, (jnp.int32(0), jnp.int32(-1), jnp.int32(0)))

    tcv = tot[pl.ds(c_sel * _L, _L)]
    suf = lax.rev(plsc.cumsum(lax.rev(tcv, (0,))), (0,))
    mask = (s_above + suf) >= need
    j_sel = jnp.sum(mask.astype(jnp.int32)) - 1
    sel = iota == j_sel
    cnt_ge = s_above + jnp.sum(jnp.where(sel, suf, 0))
    n_in = jnp.sum(jnp.where(sel, tcv, 0))
    cnt_gt = cnt_ge - n_in
    return c_sel * _L + j_sel, need - cnt_gt, n_in


def _process_row(row, cand, hist, tot, acc, iota, lane_base):
    need = jnp.int32(_K)
    ones = jnp.ones((_L,), jnp.int32)

    _clear_hist(hist)

    @plsc.parallel_loop(0, _NV0, unroll=8)
    def _hist0(i):
        x = row[pl.ds(i * _L, _L)]
        b = _byte(_keys(x), 24)
        plsc.addupdate_scatter(hist, [lane_base + b], ones)

    b0, need, _ = _select(hist, tot, need, iota)

    def body0(i, carry):
        acc, cnt16 = carry
        xs = [row[pl.ds((i * 16 + k) * _L, _L)] for k in range(16)]
        bs_ = [_byte(_keys(x), 24) for x in xs]
        for x, b in zip(xs, bs_):
            acc = acc + jnp.where(b > b0, x, 0.0)
            match = b == b0
            plsc.store_scatter(cand, [cnt16 + iota], x, mask=match)
            cnt16 = cnt16 + jnp.where(match, 16, 0)
        return acc, cnt16

    acc, cnt16 = lax.fori_loop(
        0, _NV0 // 16, body0, (acc, jnp.zeros((_L,), jnp.int32)))
    cnt = cnt16 >> 4

    src, dst = cand, row
    u_t = b0 << 24
    for lvl in (1, 2):
        shift = 24 - 8 * lvl
        mv = jnp.max(cnt)
        _clear_hist(hist)

        def bodyh(jb, carry, src=src, cnt=cnt, shift=shift):
            xs = [src[pl.ds((jb * 8 + k) * _L, _L)] for k in range(8)]
            bs_ = [_byte(_keys(x), shift) for x in xs]
            for k in range(8):
                plsc.addupdate_scatter(hist, [lane_base + bs_[k]], ones,
                                       mask=(jb * 8 + k) < cnt)
            return carry

        lax.fori_loop(0, (mv + 7) >> 3, bodyh, 0)
        bs, need, _ = _select(hist, tot, need, iota)

        def bodyc(jb, carry, src=src, dst=dst, cnt=cnt, shift=shift, bs=bs):
            acc, cnt16 = carry
            xs = [src[pl.ds((jb * 8 + k) * _L, _L)] for k in range(8)]
            bs_ = [_byte(_keys(x), shift) for x in xs]
            for k in range(8):
                x, b = xs[k], bs_[k]
                valid = (jb * 8 + k) < cnt
                acc = acc + jnp.where(jnp.logical_and(valid, b > bs), x, 0.0)
                match = jnp.logical_and(valid, b == bs)
                plsc.store_scatter(dst, [cnt16 + iota], x, mask=match)
                cnt16 = cnt16 + jnp.where(match, 16, 0)
            return acc, cnt16

        acc, cnt16 = lax.fori_loop(
            0, (mv + 7) >> 3, bodyc, (acc, jnp.zeros((_L,), jnp.int32)))
        cnt = cnt16 >> 4
        u_t = u_t | (bs << shift)
        src, dst = dst, src

    mv = jnp.max(cnt)
    _clear_hist(hist)

    def bodyh3(jb, carry, src=src, cnt=cnt):
        xs = [src[pl.ds((jb * 8 + k) * _L, _L)] for k in range(8)]
        bs_ = [_byte(_keys(x), 0) for x in xs]
        for k in range(8):
            plsc.addupdate_scatter(hist, [lane_base + bs_[k]], ones,
                                   mask=(jb * 8 + k) < cnt)
        return carry

    lax.fori_loop(0, (mv + 7) >> 3, bodyh3, 0)
    b3, need, _ = _select(hist, tot, need, iota)

    def body3(jb, acc, src=src, cnt=cnt, b3=b3):
        xs = [src[pl.ds((jb * 8 + k) * _L, _L)] for k in range(8)]
        bs_ = [_byte(_keys(x), 0) for x in xs]
        for k in range(8):
            above = jnp.logical_and((jb * 8 + k) < cnt, bs_[k] > b3)
            acc = acc + jnp.where(above, xs[k], 0.0)
        return acc

    acc = lax.fori_loop(0, (mv + 7) >> 3, body3, acc)

    u_t = u_t | b3
    uv = jnp.zeros((_L,), jnp.int32) + u_t
    iv = jnp.where(uv < 0, uv ^ _MIN32, ~uv)
    tv = lax.bitcast_convert_type(iv, jnp.float32)
    needf = (jnp.zeros((_L,), jnp.int32) + need).astype(jnp.float32)
    return acc + jnp.where(iota == 0, needf * tv, 0.0)


def _sc_body(loss_hbm, out_hbm, row0, cand, hist, tot, accv, sem0):
    wid = lax.axis_index("s") * _NC + lax.axis_index("c")
    iota = lax.iota(jnp.int32, _L)
    lane_base = iota * _PITCH

    pltpu.make_async_copy(loss_hbm.at[wid], row0.at[pl.ds(0, _P)], sem0).start()

    acc = jnp.zeros((_L,), jnp.float32)
    pltpu.make_async_copy(loss_hbm.at[wid], row0.at[pl.ds(0, _P)], sem0).wait()
    acc = _process_row(row0, cand, hist, tot, acc, iota, lane_base)

    accv[...] = acc
    pltpu.sync_copy(accv, out_hbm.at[wid])


@functools.partial(
    pl.kernel,
    out_type=jax.ShapeDtypeStruct((_NW, _L), jnp.float32),
    mesh=plsc.VectorSubcoreMesh(core_axis_name="c", subcore_axis_name="s"),
    compiler_params=pltpu.CompilerParams(needs_layout_passes=False),
    scratch_types=[
        pltpu.VMEM((_P + 128,), jnp.float32),
        pltpu.VMEM((_P + 128,), jnp.float32),
        pltpu.VMEM((_L * _PITCH,), jnp.int32),
        pltpu.VMEM((256,), jnp.int32),
        pltpu.VMEM((_L,), jnp.float32),
        pltpu.SemaphoreType.DMA,
    ],
)
def _sc_kernel(loss_hbm, out_hbm, *scratch):
    _sc_body(loss_hbm, out_hbm, *scratch)


def _tc_body(x_ref, out_ref):
    x = x_ref[...]
    i = lax.bitcast_convert_type(x, jnp.int32)
    s = i ^ ((i >> 31) & jnp.int32(0x7FFFFFFF))

    def step(_, lohi):
        lo, hi = lohi
        mid = (lo >> 1) + (hi >> 1) + (lo & hi & 1) + ((lo ^ hi) & 1)
        cnt = jnp.sum((s >= mid).astype(jnp.int32), axis=1, keepdims=True)
        pred = cnt >= _K
        lo = jnp.where(pred, mid, lo)
        hi = jnp.where(pred, hi, mid - 1)
        return lo, hi

    nrows = x.shape[0]
    lo0 = jnp.full((nrows, 1), jnp.iinfo(jnp.int32).min, jnp.int32)
    hi0 = jnp.full((nrows, 1), jnp.iinfo(jnp.int32).max, jnp.int32)
    t, _ = jax.lax.fori_loop(0, 32, step, (lo0, hi0))

    gt = s > t
    cnt_gt = jnp.sum(gt.astype(jnp.float32), axis=1, keepdims=True)
    sum_gt = jnp.sum(jnp.where(gt, x, 0.0), axis=1, keepdims=True)
    tf = lax.bitcast_convert_type(t ^ ((t >> 31) & jnp.int32(0x7FFFFFFF)),
                                  jnp.float32)
    row = sum_gt + (jnp.float32(_K) - cnt_gt) * tf
    out_ref[0, 0] = jnp.sum(row)


def _tc_sum(x):
    return pl.pallas_call(
        _tc_body,
        grid=(1,),
        out_shape=jax.ShapeDtypeStruct((1, 1), jnp.float32),
        in_specs=[pl.BlockSpec((_B - _NW, _P), lambda i: (1, 0))],
        out_specs=pl.BlockSpec((1, 1), lambda i: (0, 0),
                               memory_space=pltpu.SMEM),
    )(x)


def kernel(loss, dummy):
    sc_part = _sc_kernel(loss)
    tc_part = _tc_sum(loss)
    return (jnp.sum(sc_part) + tc_part[0, 0]) / jnp.float32(_B * _K)


# speedup vs baseline: 1.3071x; 1.0187x over previous
import functools

import jax
import jax.numpy as jnp
import numpy as np
from jax import lax
from jax.experimental import pallas as pl
from jax.experimental.pallas import tpu as pltpu
from jax.experimental.pallas import tpu_sc as plsc

_B = 64
_P = 32768
_K = 8192
_L = 16
_NC = 2
_NS = 16
_NW = _NC * _NS
_NV0 = _P // _L
_PITCH = 257
_MIN32 = np.int32(-2147483648)


def _keys(x):
    i = lax.bitcast_convert_type(x, jnp.int32)
    return i ^ ((i >> 31) | _MIN32)


def _byte(u, shift):
    return lax.shift_right_logical(u, shift) & 255


def _clear_hist(hist):
    zeros = jnp.zeros((_L,), jnp.int32)

    def clr(j, carry):
        hist[pl.ds(j * _L, _L)] = zeros
        return carry

    lax.fori_loop(0, _PITCH, clr, 0, unroll=16)


def _select(hist, tot, need, iota):
    for c in range(16):
        tc = hist[pl.ds(c * _L, _L)]
        for lane in range(1, _L):
            tc = tc + hist[pl.ds(lane * _PITCH + c * _L, _L)]
        tot[pl.ds(c * _L, _L)] = tc

    def cscan(i, carry):
        s_run, c_sel, s_above = carry
        cc = 15 - i
        csum = jnp.sum(tot[pl.ds(cc * _L, _L)])
        s_new = s_run + csum
        hit = jnp.logical_and(c_sel < 0, s_new >= need)
        c_sel = jnp.where(hit, cc, c_sel)
        s_above = jnp.where(hit, s_run, s_above)
        return s_new, c_sel, s_above

    _, c_sel, s_above = lax.fori_loop(
        0, 16, cscan, (jnp.int32(0), jnp.int32(-1), jnp.int32(0)))

    tcv = tot[pl.ds(c_sel * _L, _L)]
    suf = lax.rev(plsc.cumsum(lax.rev(tcv, (0,))), (0,))
    mask = (s_above + suf) >= need
    j_sel = jnp.sum(mask.astype(jnp.int32)) - 1
    sel = iota == j_sel
    cnt_ge = s_above + jnp.sum(jnp.where(sel, suf, 0))
    n_in = jnp.sum(jnp.where(sel, tcv, 0))
    cnt_gt = cnt_ge - n_in
    return c_sel * _L + j_sel, need - cnt_gt, n_in


def _process_row(row, cand, hist, tot, acc, iota, lane_base):
    need = jnp.int32(_K)
    ones = jnp.ones((_L,), jnp.int32)

    _clear_hist(hist)

    @plsc.parallel_loop(0, _NV0, unroll=8)
    def _hist0(i):
        x = row[pl.ds(i * _L, _L)]
        b = _byte(_keys(x), 24)
        plsc.addupdate_scatter(hist, [lane_base + b], ones)

    b0, need, _ = _select(hist, tot, need, iota)

    @plsc.parallel_loop(0, _NV0, unroll=8,
                        carry=(acc, jnp.zeros((_L,), jnp.int32)))
    def _compact0(i, carry):
        acc, cnt16 = carry
        x = row[pl.ds(i * _L, _L)]
        b = _byte(_keys(x), 24)
        acc = acc + jnp.where(b > b0, x, 0.0)
        match = b == b0
        plsc.store_scatter(cand, [cnt16 + iota], x, mask=match)
        cnt16 = cnt16 + jnp.where(match, 16, 0)
        return acc, cnt16

    acc, cnt16 = _compact0
    cnt = cnt16 >> 4

    src, dst = cand, row
    u_t = b0 << 24
    for lvl in (1, 2):
        shift = 24 - 8 * lvl
        mv = jnp.max(cnt)
        _clear_hist(hist)

        def bodyh(jb, carry, src=src, cnt=cnt, shift=shift):
            xs = [src[pl.ds((jb * 8 + k) * _L, _L)] for k in range(8)]
            bs_ = [_byte(_keys(x), shift) for x in xs]
            for k in range(8):
                plsc.addupdate_scatter(hist, [lane_base + bs_[k]], ones,
                                       mask=(jb * 8 + k) < cnt)
            return carry

        lax.fori_loop(0, (mv + 7) >> 3, bodyh, 0)
        bs, need, _ = _select(hist, tot, need, iota)

        def bodyc(jb, carry, src=src, dst=dst, cnt=cnt, shift=shift, bs=bs):
            acc, cnt16 = carry
            xs = [src[pl.ds((jb * 8 + k) * _L, _L)] for k in range(8)]
            bs_ = [_byte(_keys(x), shift) for x in xs]
            for k in range(8):
                x, b = xs[k], bs_[k]
                valid = (jb * 8 + k) < cnt
                acc = acc + jnp.where(jnp.logical_and(valid, b > bs), x, 0.0)
                match = jnp.logical_and(valid, b == bs)
                plsc.store_scatter(dst, [cnt16 + iota], x, mask=match)
                cnt16 = cnt16 + jnp.where(match, 16, 0)
            return acc, cnt16

        acc, cnt16 = lax.fori_loop(
            0, (mv + 7) >> 3, bodyc, (acc, jnp.zeros((_L,), jnp.int32)))
        cnt = cnt16 >> 4
        u_t = u_t | (bs << shift)
        src, dst = dst, src

    mv = jnp.max(cnt)
    _clear_hist(hist)

    def bodyh3(jb, carry, src=src, cnt=cnt):
        xs = [src[pl.ds((jb * 8 + k) * _L, _L)] for k in range(8)]
        bs_ = [_byte(_keys(x), 0) for x in xs]
        for k in range(8):
            plsc.addupdate_scatter(hist, [lane_base + bs_[k]], ones,
                                   mask=(jb * 8 + k) < cnt)
        return carry

    lax.fori_loop(0, (mv + 7) >> 3, bodyh3, 0)
    b3, need, _ = _select(hist, tot, need, iota)

    def body3(jb, acc, src=src, cnt=cnt, b3=b3):
        xs = [src[pl.ds((jb * 8 + k) * _L, _L)] for k in range(8)]
        bs_ = [_byte(_keys(x), 0) for x in xs]
        for k in range(8):
            above = jnp.logical_and((jb * 8 + k) < cnt, bs_[k] > b3)
            acc = acc + jnp.where(above, xs[k], 0.0)
        return acc

    acc = lax.fori_loop(0, (mv + 7) >> 3, body3, acc)

    u_t = u_t | b3
    uv = jnp.zeros((_L,), jnp.int32) + u_t
    iv = jnp.where(uv < 0, uv ^ _MIN32, ~uv)
    tv = lax.bitcast_convert_type(iv, jnp.float32)
    needf = (jnp.zeros((_L,), jnp.int32) + need).astype(jnp.float32)
    return acc + jnp.where(iota == 0, needf * tv, 0.0)


def _sc_body(loss_hbm, out_hbm, row0, cand, hist, tot, accv, sem0):
    wid = lax.axis_index("s") * _NC + lax.axis_index("c")
    iota = lax.iota(jnp.int32, _L)
    lane_base = iota * _PITCH

    pltpu.make_async_copy(loss_hbm.at[wid], row0.at[pl.ds(0, _P)], sem0).start()

    acc = jnp.zeros((_L,), jnp.float32)
    pltpu.make_async_copy(loss_hbm.at[wid], row0.at[pl.ds(0, _P)], sem0).wait()
    acc = _process_row(row0, cand, hist, tot, acc, iota, lane_base)

    accv[...] = acc
    pltpu.sync_copy(accv, out_hbm.at[wid])


@functools.partial(
    pl.kernel,
    out_type=jax.ShapeDtypeStruct((_NW, _L), jnp.float32),
    mesh=plsc.VectorSubcoreMesh(core_axis_name="c", subcore_axis_name="s"),
    compiler_params=pltpu.CompilerParams(needs_layout_passes=False),
    scratch_types=[
        pltpu.VMEM((_P + 128,), jnp.float32),
        pltpu.VMEM((_P + 128,), jnp.float32),
        pltpu.VMEM((_L * _PITCH,), jnp.int32),
        pltpu.VMEM((256,), jnp.int32),
        pltpu.VMEM((_L,), jnp.float32),
        pltpu.SemaphoreType.DMA,
    ],
)
def _sc_kernel(loss_hbm, out_hbm, *scratch):
    _sc_body(loss_hbm, out_hbm, *scratch)


def _tc_body(x_ref, out_ref):
    x = x_ref[...]
    i = lax.bitcast_convert_type(x, jnp.int32)
    s = i ^ ((i >> 31) & jnp.int32(0x7FFFFFFF))

    def step(_, lohi):
        lo, hi = lohi
        mid = (lo >> 1) + (hi >> 1) + (lo & hi & 1) + ((lo ^ hi) & 1)
        cnt = jnp.sum((s >= mid).astype(jnp.int32), axis=1, keepdims=True)
        pred = cnt >= _K
        lo = jnp.where(pred, mid, lo)
        hi = jnp.where(pred, hi, mid - 1)
        return lo, hi

    nrows = x.shape[0]
    lo0 = jnp.full((nrows, 1), jnp.iinfo(jnp.int32).min, jnp.int32)
    hi0 = jnp.full((nrows, 1), jnp.iinfo(jnp.int32).max, jnp.int32)
    t, _ = jax.lax.fori_loop(0, 32, step, (lo0, hi0))

    gt = s > t
    cnt_gt = jnp.sum(gt.astype(jnp.float32), axis=1, keepdims=True)
    sum_gt = jnp.sum(jnp.where(gt, x, 0.0), axis=1, keepdims=True)
    tf = lax.bitcast_convert_type(t ^ ((t >> 31) & jnp.int32(0x7FFFFFFF)),
                                  jnp.float32)
    row = sum_gt + (jnp.float32(_K) - cnt_gt) * tf
    out_ref[0, 0] = jnp.sum(row)


def _tc_sum(x):
    return pl.pallas_call(
        _tc_body,
        grid=(1,),
        out_shape=jax.ShapeDtypeStruct((1, 1), jnp.float32),
        in_specs=[pl.BlockSpec((_B - _NW, _P), lambda i: (1, 0))],
        out_specs=pl.BlockSpec((1, 1), lambda i: (0, 0),
                               memory_space=pltpu.SMEM),
    )(x)


def kernel(loss, dummy):
    sc_part = _sc_kernel(loss)
    tc_part = _tc_sum(loss)
    return (jnp.sum(sc_part) + tc_part[0, 0]) / jnp.float32(_B * _K)


# speedup vs baseline: 1.3135x; 1.0049x over previous
import functools

import jax
import jax.numpy as jnp
import numpy as np
from jax import lax
from jax.experimental import pallas as pl
from jax.experimental.pallas import tpu as pltpu
from jax.experimental.pallas import tpu_sc as plsc

_B = 64
_P = 32768
_K = 8192
_L = 16
_NC = 2
_NS = 16
_NW = _NC * _NS
_NV0 = _P // _L
_PITCH = 257
_MIN32 = np.int32(-2147483648)


def _keys(x):
    i = lax.bitcast_convert_type(x, jnp.int32)
    return i ^ ((i >> 31) | _MIN32)


def _byte(u, shift):
    return lax.shift_right_logical(u, shift) & 255


def _clear_hist(hist):
    zeros = jnp.zeros((_L,), jnp.int32)

    def clr(j, carry):
        hist[pl.ds(j * _L, _L)] = zeros
        return carry

    lax.fori_loop(0, _PITCH, clr, 0, unroll=16)


def _select(hist, tot, need, iota):
    for c in range(16):
        tc = hist[pl.ds(c * _L, _L)]
        for lane in range(1, _L):
            tc = tc + hist[pl.ds(lane * _PITCH + c * _L, _L)]
        tot[pl.ds(c * _L, _L)] = tc

    def cscan(i, carry):
        s_run, c_sel, s_above = carry
        cc = 15 - i
        csum = jnp.sum(tot[pl.ds(cc * _L, _L)])
        s_new = s_run + csum
        hit = jnp.logical_and(c_sel < 0, s_new >= need)
        c_sel = jnp.where(hit, cc, c_sel)
        s_above = jnp.where(hit, s_run, s_above)
        return s_new, c_sel, s_above

    _, c_sel, s_above = lax.fori_loop(
        0, 16, cscan, (jnp.int32(0), jnp.int32(-1), jnp.int32(0)))

    tcv = tot[pl.ds(c_sel * _L, _L)]
    suf = lax.rev(plsc.cumsum(lax.rev(tcv, (0,))), (0,))
    mask = (s_above + suf) >= need
    j_sel = jnp.sum(mask.astype(jnp.int32)) - 1
    sel = iota == j_sel
    cnt_ge = s_above + jnp.sum(jnp.where(sel, suf, 0))
    n_in = jnp.sum(jnp.where(sel, tcv, 0))
    cnt_gt = cnt_ge - n_in
    return c_sel * _L + j_sel, need - cnt_gt, n_in


def _process_row(row, cand, hist, tot, acc, iota, lane_base):
    need = jnp.int32(_K)
    ones = jnp.ones((_L,), jnp.int32)

    _clear_hist(hist)

    @plsc.parallel_loop(0, _NV0, unroll=8)
    def _hist0(i):
        x = row[pl.ds(i * _L, _L)]
        b = _byte(_keys(x), 24)
        plsc.addupdate_scatter(hist, [lane_base + b], ones)

    b0, need, _ = _select(hist, tot, need, iota)

    @plsc.parallel_loop(0, _NV0, unroll=8,
                        carry=(acc, jnp.zeros((_L,), jnp.int32)))
    def _compact0(i, carry):
        acc, cnt16 = carry
        x = row[pl.ds(i * _L, _L)]
        b = _byte(_keys(x), 24)
        acc = acc + jnp.where(b > b0, x, 0.0)
        match = b == b0
        plsc.store_scatter(cand, [cnt16 + iota], x, mask=match)
        cnt16 = cnt16 + jnp.where(match, 16, 0)
        return acc, cnt16

    acc, cnt16 = _compact0
    cnt = cnt16 >> 4

    src, dst = cand, row
    u_t = b0 << 24
    for lvl in (1, 2):
        shift = 24 - 8 * lvl
        mv = jnp.max(cnt)
        _clear_hist(hist)

        def bodyh(jb, carry, src=src, cnt=cnt, shift=shift):
            xs = [src[pl.ds((jb * 8 + k) * _L, _L)] for k in range(8)]
            bs_ = [_byte(_keys(x), shift) for x in xs]
            for k in range(8):
                plsc.addupdate_scatter(hist, [lane_base + bs_[k]], ones,
                                       mask=(jb * 8 + k) < cnt)
            return carry

        lax.fori_loop(0, (mv + 7) >> 3, bodyh, 0)
        bs, need, _ = _select(hist, tot, need, iota)

        def bodyc(jb, carry, src=src, dst=dst, cnt=cnt, shift=shift, bs=bs):
            acc, cnt16 = carry
            xs = [src[pl.ds((jb * 8 + k) * _L, _L)] for k in range(8)]
            bs_ = [_byte(_keys(x), shift) for x in xs]
            for k in range(8):
                x, b = xs[k], bs_[k]
                valid = (jb * 8 + k) < cnt
                acc = acc + jnp.where(jnp.logical_and(valid, b > bs), x, 0.0)
                match = jnp.logical_and(valid, b == bs)
                plsc.store_scatter(dst, [cnt16 + iota], x, mask=match)
                cnt16 = cnt16 + jnp.where(match, 16, 0)
            return acc, cnt16

        acc, cnt16 = lax.fori_loop(
            0, (mv + 7) >> 3, bodyc, (acc, jnp.zeros((_L,), jnp.int32)))
        cnt = cnt16 >> 4
        u_t = u_t | (bs << shift)
        src, dst = dst, src

    mv = jnp.max(cnt)
    _clear_hist(hist)

    def bodyh3(jb, carry, src=src, cnt=cnt):
        xs = [src[pl.ds((jb * 8 + k) * _L, _L)] for k in range(8)]
        bs_ = [_byte(_keys(x), 0) for x in xs]
        for k in range(8):
            plsc.addupdate_scatter(hist, [lane_base + bs_[k]], ones,
                                   mask=(jb * 8 + k) < cnt)
        return carry

    lax.fori_loop(0, (mv + 7) >> 3, bodyh3, 0)
    b3, need, _ = _select(hist, tot, need, iota)

    def body3(jb, acc, src=src, cnt=cnt, b3=b3):
        xs = [src[pl.ds((jb * 8 + k) * _L, _L)] for k in range(8)]
        bs_ = [_byte(_keys(x), 0) for x in xs]
        for k in range(8):
            above = jnp.logical_and((jb * 8 + k) < cnt, bs_[k] > b3)
            acc = acc + jnp.where(above, xs[k], 0.0)
        return acc

    acc = lax.fori_loop(0, (mv + 7) >> 3, body3, acc)

    u_t = u_t | b3
    uv = jnp.zeros((_L,), jnp.int32) + u_t
    iv = jnp.where(uv < 0, uv ^ _MIN32, ~uv)
    tv = lax.bitcast_convert_type(iv, jnp.float32)
    needf = (jnp.zeros((_L,), jnp.int32) + need).astype(jnp.float32)
    return acc + jnp.where(iota == 0, needf * tv, 0.0)


def _sc_body(loss_hbm, out_hbm, row0, cand, hist, tot, accv, sem0):
    wid = lax.axis_index("s") * _NC + lax.axis_index("c")
    iota = lax.iota(jnp.int32, _L)
    lane_base = iota * _PITCH

    pltpu.make_async_copy(loss_hbm.at[wid], row0.at[pl.ds(0, _P)], sem0).start()

    acc = jnp.zeros((_L,), jnp.float32)
    pltpu.make_async_copy(loss_hbm.at[wid], row0.at[pl.ds(0, _P)], sem0).wait()
    acc = _process_row(row0, cand, hist, tot, acc, iota, lane_base)

    accv[...] = acc
    pltpu.sync_copy(accv, out_hbm.at[wid])


@functools.partial(
    pl.kernel,
    out_type=jax.ShapeDtypeStruct((_NW, _L), jnp.float32),
    mesh=plsc.VectorSubcoreMesh(core_axis_name="c", subcore_axis_name="s"),
    compiler_params=pltpu.CompilerParams(needs_layout_passes=False),
    scratch_types=[
        pltpu.VMEM((_P + 128,), jnp.float32),
        pltpu.VMEM((_P + 128,), jnp.float32),
        pltpu.VMEM((_L * _PITCH,), jnp.int32),
        pltpu.VMEM((256,), jnp.int32),
        pltpu.VMEM((_L,), jnp.float32),
        pltpu.SemaphoreType.DMA,
    ],
)
def _sc_kernel(loss_hbm, out_hbm, *scratch):
    _sc_body(loss_hbm, out_hbm, *scratch)


def _tc_body(x_ref, out_ref):
    x = x_ref[...]
    i = lax.bitcast_convert_type(x, jnp.int32)
    s = i ^ ((i >> 31) & jnp.int32(0x7FFFFFFF))

    def step(_, lohi):
        lo, hi = lohi
        mid = (lo >> 1) + (hi >> 1) + (lo & hi & 1) + ((lo ^ hi) & 1)
        cnt = jnp.sum((s >= mid).astype(jnp.int32), axis=1, keepdims=True)
        pred = cnt >= _K
        lo = jnp.where(pred, mid, lo)
        hi = jnp.where(pred, hi, mid - 1)
        return lo, hi

    lo0 = jnp.min(s, axis=1, keepdims=True)
    hi0 = jnp.max(s, axis=1, keepdims=True)
    t, _ = jax.lax.while_loop(
        lambda lohi: jnp.any(lohi[0] < lohi[1]),
        lambda lohi: step(0, lohi),
        (lo0, hi0))

    gt = s > t
    cnt_gt = jnp.sum(gt.astype(jnp.float32), axis=1, keepdims=True)
    sum_gt = jnp.sum(jnp.where(gt, x, 0.0), axis=1, keepdims=True)
    tf = lax.bitcast_convert_type(t ^ ((t >> 31) & jnp.int32(0x7FFFFFFF)),
                                  jnp.float32)
    row = sum_gt + (jnp.float32(_K) - cnt_gt) * tf
    out_ref[0, 0] = jnp.sum(row)


def _tc_sum(x):
    return pl.pallas_call(
        _tc_body,
        grid=(1,),
        out_shape=jax.ShapeDtypeStruct((1, 1), jnp.float32),
        in_specs=[pl.BlockSpec((_B - _NW, _P), lambda i: (1, 0))],
        out_specs=pl.BlockSpec((1, 1), lambda i: (0, 0),
                               memory_space=pltpu.SMEM),
    )(x)


def kernel(loss, dummy):
    sc_part = _sc_kernel(loss)
    tc_part = _tc_sum(loss)
    return (jnp.sum(sc_part) + tc_part[0, 0]) / jnp.float32(_B * _K)
